# Initial kernel scaffold; baseline (speedup 1.0000x reference)
import jax
import jax.numpy as jnp
from jax.experimental import pallas as pl


def kernel(x, pos_table, alpha, pe):
    raise NotImplementedError("write your pallas kernel here")


# speedup vs baseline: 1.8449x; 1.8449x over previous
import jax
import jax.numpy as jnp
from jax.experimental import pallas as pl
from jax.experimental.pallas import tpu as pltpu


def _blend_kernel(a_ref, x_ref, pt_ref, pe_ref, o_ref):
    a = jax.nn.sigmoid(a_ref[0, 0])
    blend = a * pt_ref[...] + (1.0 - a) * pe_ref[...]
    o_ref[...] = x_ref[...] + blend[None, :, :]


def kernel(x, pos_table, alpha, pe):
    batch, seq_len, d_model = x.shape
    pt = pos_table[:seq_len]
    fpe = pe[:seq_len]
    a2 = jnp.reshape(alpha, (1, 1))

    block_s = 512
    grid = (seq_len // block_s,)

    return pl.pallas_call(
        _blend_kernel,
        grid=grid,
        in_specs=[
            pl.BlockSpec(memory_space=pltpu.SMEM),
            pl.BlockSpec((batch, block_s, d_model), lambda i: (0, i, 0)),
            pl.BlockSpec((block_s, d_model), lambda i: (i, 0)),
            pl.BlockSpec((block_s, d_model), lambda i: (i, 0)),
        ],
        out_specs=pl.BlockSpec((batch, block_s, d_model), lambda i: (0, i, 0)),
        out_shape=jax.ShapeDtypeStruct((batch, seq_len, d_model), jnp.float32),
    )(a2, x, pt, fpe)
